# baseline (speedup 1.0000x reference)
import jax
import jax.numpy as jnp
from jax import lax
from jax.experimental import pallas as pl
from jax.experimental.pallas import tpu as pltpu


def _deint_kernel(x_ref, o_ref):
    v = x_ref[0, 0].reshape(1024, 128)
    ss = lax.broadcasted_iota(jnp.int32, (128, 64), 0)
    ll = lax.broadcasted_iota(jnp.int32, (128, 64), 1)
    for j in range(2):
        fj = jnp.logical_and(
            ss // 64 == j,
            ll == 2 * (ss % 32) + (ss // 32) % 2,
        ).astype(v.dtype)
        g = jnp.dot(v, fj, preferred_element_type=jnp.float32)
        o_ref[0, 0, :, 0:1, pl.Slice(j, 32, 2), :] = g.reshape(32, 1, 32, 64)


def kernel(x):
    B, Cr3, H, W, Z = x.shape
    C = Cr3 // 8
    xv = (
        x.reshape(B, 2 * C, 4, H, W, Z)
        .transpose(0, 1, 3, 4, 2, 5)
        .reshape(B, 2 * C, H, W, 4 * Z)
    )
    out = pl.pallas_call(
        _deint_kernel,
        grid=(B, C, 2),
        in_specs=[
            pl.BlockSpec(
                (1, 1, H, W, 4 * Z),
                lambda b, c, i: (b, 2 * c + i, 0, 0, 0),
            )
        ],
        out_specs=pl.BlockSpec(
            (1, 1, H, 1, 2 * W, 2 * Z),
            lambda b, c, i: (b, c, 0, i, 0, 0),
        ),
        out_shape=jax.ShapeDtypeStruct((B, C, H, 2, 2 * W, 2 * Z), x.dtype),
        compiler_params=pltpu.CompilerParams(
            dimension_semantics=("parallel", "parallel", "parallel"),
        ),
    )(xv)
    return out.reshape(B, C, 2 * H, 2 * W, 2 * Z)


# speedup vs baseline: 1.8467x; 1.8467x over previous
import jax
import jax.numpy as jnp
from jax import lax
from jax.experimental import pallas as pl
from jax.experimental.pallas import tpu as pltpu

_CB = 4


def _deint_kernel(x_ref, o_ref):
    v = x_ref[0]
    cb = v.shape[0]
    ss = lax.broadcasted_iota(jnp.int32, (64, 64), 0)
    ll = lax.broadcasted_iota(jnp.int32, (64, 64), 1)
    g2 = (ll == 2 * (ss % 32) + ss // 32).astype(v.dtype)
    for i in range(2):
        for j in range(2):
            a = jnp.concatenate(
                [v[:, 4 * i + 2 * j].reshape(cb * 1024, 32),
                 v[:, 4 * i + 2 * j + 1].reshape(cb * 1024, 32)],
                axis=1,
            )
            g = jnp.dot(a, g2, preferred_element_type=jnp.float32)
            o_ref[0, :, :, i : i + 1, pl.Slice(j, 32, 2), :] = (
                g.reshape(cb, 32, 1, 32, 64))


def kernel(x):
    B, Cr3, H, W, Z = x.shape
    C = Cr3 // 8
    xr = x.reshape(B, C, 8, H, W, Z)
    out = pl.pallas_call(
        _deint_kernel,
        grid=(B, C // _CB),
        in_specs=[
            pl.BlockSpec(
                (1, _CB, 8, H, W, Z),
                lambda b, c: (b, c, 0, 0, 0, 0),
            )
        ],
        out_specs=pl.BlockSpec(
            (1, _CB, H, 2, 2 * W, 2 * Z),
            lambda b, c: (b, c, 0, 0, 0, 0),
        ),
        out_shape=jax.ShapeDtypeStruct((B, C, H, 2, 2 * W, 2 * Z), x.dtype),
        compiler_params=pltpu.CompilerParams(
            dimension_semantics=("parallel", "parallel"),
        ),
    )(xr)
    return out.reshape(B, C, 2 * H, 2 * W, 2 * Z)
